# Initial kernel scaffold; baseline (speedup 1.0000x reference)
import jax
import jax.numpy as jnp
from jax.experimental import pallas as pl


def kernel(x, edge_index, W0, b0, g0, be0, W1, b1, g1, be1, W2, b2, g2, be2):
    raise NotImplementedError("write your pallas kernel here")


# speedup vs baseline: 1.0100x; 1.0100x over previous
import functools

import jax
import jax.numpy as jnp
from jax.experimental import pallas as pl

N_NODES = 10000
N_EDGES = 320000
DELTA = 4.0
ROW_BLK = 400


def _dense_body(amax_ref, amin_ref, s1_ref, s2_ref, cnt_ref, w_ref, b_ref,
                g_ref, be_ref, out_ref):
    amax = amax_ref[...]
    amin = amin_ref[...]
    s1 = s1_ref[...]
    s2 = s2_ref[...]
    cnt = cnt_ref[...]
    mean = s1 / cnt
    var = jnp.maximum(s2 / cnt - mean * mean, 0.0)
    std = jnp.sqrt(var + 1e-5)
    aggs = jnp.concatenate([amax, amin, std, var], axis=1)
    logd = jnp.log(cnt + 1.0)
    t1 = logd / DELTA
    t2 = DELTA / logd
    scaled = jnp.concatenate([aggs, aggs * t1[:, :1], aggs * t2[:, :1]], axis=1)
    h = jax.lax.dot_general(scaled, w_ref[...], (((1,), (0,)), ((), ())),
                            preferred_element_type=jnp.float32)
    h = h + b_ref[...]
    mu = jnp.mean(h, axis=-1, keepdims=True)
    v = jnp.var(h, axis=-1, keepdims=True)
    h = (h - mu) / jnp.sqrt(v + 1e-5) * g_ref[...] + be_ref[...]
    out_ref[...] = jnp.where(h > 0, h, jnp.exp(h) - 1.0)


@functools.partial(jax.jit, static_argnames=())
def _dense_layer(amax, amin, s1, s2, cnt2d, W, b, g, be):
    grid = (N_NODES // ROW_BLK,)
    node_spec = pl.BlockSpec((ROW_BLK, 128), lambda i: (i, 0))
    return pl.pallas_call(
        _dense_body,
        grid=grid,
        in_specs=[
            node_spec, node_spec, node_spec, node_spec, node_spec,
            pl.BlockSpec((W.shape[0], 128), lambda i: (0, 0)),
            pl.BlockSpec((1, 128), lambda i: (0, 0)),
            pl.BlockSpec((1, 128), lambda i: (0, 0)),
            pl.BlockSpec((1, 128), lambda i: (0, 0)),
        ],
        out_specs=node_spec,
        out_shape=jax.ShapeDtypeStruct((N_NODES, 128), jnp.float32),
    )(amax, amin, s1, s2, cnt2d, W, b.reshape(1, 128), g.reshape(1, 128),
      be.reshape(1, 128))


def kernel(x, edge_index, W0, b0, g0, be0, W1, b1, g1, be1, W2, b2, g2, be2):
    loops = jnp.arange(N_NODES, dtype=edge_index.dtype)
    src = jnp.concatenate([edge_index[0], loops])
    dst = jnp.concatenate([edge_index[1], loops])
    deg = jax.ops.segment_sum(jnp.ones(src.shape[0], jnp.float32), dst,
                              num_segments=N_NODES)
    cnt2d = jnp.broadcast_to(deg[:, None], (N_NODES, 128))
    h = x
    for (W, b, g, be) in ((W0, b0, g0, be0), (W1, b1, g1, be1),
                          (W2, b2, g2, be2)):
        m = h[src]
        amax = jax.ops.segment_max(m, dst, num_segments=N_NODES)
        amin = -jax.ops.segment_max(-m, dst, num_segments=N_NODES)
        s1 = jax.ops.segment_sum(m, dst, num_segments=N_NODES)
        s2 = jax.ops.segment_sum(m * m, dst, num_segments=N_NODES)
        h = _dense_layer(amax, amin, s1, s2, cnt2d, W, b, g, be)
    return h
